# initial kernel scaffold (unmeasured)
import jax
import jax.numpy as jnp
from jax import lax
from jax.experimental import pallas as pl
from jax.experimental.pallas import tpu as pltpu

N_DEV = 8

_sem_signal = getattr(pl, "semaphore_signal", None) or pltpu.semaphore_signal
_sem_wait = getattr(pl, "semaphore_wait", None) or pltpu.semaphore_wait
_DeviceIdType = getattr(pl, "DeviceIdType", None) or pltpu.DeviceIdType
_MESH = _DeviceIdType.MESH
_CompilerParams = getattr(pltpu, "CompilerParams", None) or pltpu.TPUCompilerParams


def kernel(x, w_mat):
    x = x.astype(jnp.bfloat16)
    w = w_mat.astype(jnp.bfloat16)
    m_per, k = x.shape
    _, n_per = w.shape
    m_tot = N_DEV * m_per

    def body(x_ref, w_ref, out_ref, gath_ref, amax_ref,
             send_sems, recv_sems, am_send_sems, am_recv_sems):
        my = lax.axis_index("i")
        left = lax.rem(my + N_DEV - 1, N_DEV)
        right = lax.rem(my + 1, N_DEV)

        barrier = pltpu.get_barrier_semaphore()
        for nbr in (left, right):
            _sem_signal(barrier, 1, device_id=(nbr,), device_id_type=_MESH)
        _sem_wait(barrier, 2)

        gath_ref[pl.ds(my * m_per, m_per), :] = x_ref[...]
        out_ref[pl.ds(my * m_per, m_per), :] = jnp.dot(
            x_ref[...], w_ref[...], preferred_element_type=jnp.float32)

        for h in range(N_DEV - 1):
            o_s = lax.rem(my - h + N_DEV, N_DEV)
            o_r = lax.rem(my - h - 1 + N_DEV, N_DEV)
            send = pltpu.make_async_remote_copy(
                src_ref=gath_ref.at[pl.ds(o_s * m_per, m_per), :],
                dst_ref=gath_ref.at[pl.ds(o_s * m_per, m_per), :],
                send_sem=send_sems.at[h],
                recv_sem=recv_sems.at[h],
                device_id=(right,),
                device_id_type=_MESH)
            send.start()
            send.wait_send()
            recv = pltpu.make_async_remote_copy(
                src_ref=gath_ref.at[pl.ds(o_r * m_per, m_per), :],
                dst_ref=gath_ref.at[pl.ds(o_r * m_per, m_per), :],
                send_sem=send_sems.at[h],
                recv_sem=recv_sems.at[h],
                device_id=(left,),
                device_id_type=_MESH)
            recv.wait_recv()
            out_ref[pl.ds(o_r * m_per, m_per), :] = jnp.dot(
                gath_ref[pl.ds(o_r * m_per, m_per), :], w_ref[...],
                preferred_element_type=jnp.float32)

        amax_local = jnp.max(jnp.abs(out_ref[...]))
        amax_ref[pl.ds(my, 1), :] = jnp.broadcast_to(
            amax_local, (1, 128)).astype(jnp.float32)
        am_sends = []
        for d in range(1, N_DEV):
            tgt = lax.rem(my + d, N_DEV)
            s = pltpu.make_async_remote_copy(
                src_ref=amax_ref.at[pl.ds(my, 1), :],
                dst_ref=amax_ref.at[pl.ds(my, 1), :],
                send_sem=am_send_sems.at[d - 1],
                recv_sem=am_recv_sems.at[d - 1],
                device_id=(tgt,),
                device_id_type=_MESH)
            s.start()
            am_sends.append(s)
        for d in range(1, N_DEV):
            src = lax.rem(my - d + N_DEV, N_DEV)
            r = pltpu.make_async_remote_copy(
                src_ref=amax_ref.at[pl.ds(src, 1), :],
                dst_ref=amax_ref.at[pl.ds(src, 1), :],
                send_sem=am_send_sems.at[d - 1],
                recv_sem=am_recv_sems.at[d - 1],
                device_id=(src,),
                device_id_type=_MESH)
            r.wait_recv()
        for s in am_sends:
            s.wait_send()

        amax_g = jnp.max(amax_ref[...])
        inv = 448.0 / amax_g
        scale = amax_g / 448.0
        y = out_ref[...]
        q = jnp.clip(y * inv, -448.0, 448.0).astype(jnp.float8_e4m3fn)
        out_ref[...] = q.astype(jnp.float32) * scale

    return pl.pallas_call(
        body,
        out_shape=jax.ShapeDtypeStruct((m_tot, n_per), jnp.float32),
        in_specs=[pl.BlockSpec(memory_space=pltpu.VMEM),
                  pl.BlockSpec(memory_space=pltpu.VMEM)],
        out_specs=pl.BlockSpec(memory_space=pltpu.VMEM),
        scratch_shapes=[
            pltpu.VMEM((m_tot, k), jnp.bfloat16),
            pltpu.VMEM((N_DEV, 128), jnp.float32),
            pltpu.SemaphoreType.DMA((N_DEV - 1,)),
            pltpu.SemaphoreType.DMA((N_DEV - 1,)),
            pltpu.SemaphoreType.DMA((N_DEV - 1,)),
            pltpu.SemaphoreType.DMA((N_DEV - 1,)),
        ],
        compiler_params=_CompilerParams(collective_id=0),
    )(x, w)


# baseline (device time: 414477 ns/iter reference)
import jax
import jax.numpy as jnp
from jax import lax
from jax.experimental import pallas as pl
from jax.experimental.pallas import tpu as pltpu

N_DEV = 8

_sem_signal = getattr(pl, "semaphore_signal", None) or pltpu.semaphore_signal
_sem_wait = getattr(pl, "semaphore_wait", None) or pltpu.semaphore_wait
_DeviceIdType = getattr(pl, "DeviceIdType", None) or pltpu.DeviceIdType
_MESH = _DeviceIdType.MESH
_CompilerParams = getattr(pltpu, "CompilerParams", None) or pltpu.TPUCompilerParams
_ANY = getattr(pl, "ANY", None) or getattr(pltpu, "ANY", None) or (
    pltpu.MemorySpace.ANY if hasattr(pltpu, "MemorySpace")
    else pltpu.TPUMemorySpace.ANY)


def kernel(x, w_mat):
    x = x.astype(jnp.bfloat16)
    w = w_mat.astype(jnp.bfloat16)
    m_per, k = x.shape
    _, n_per = w.shape
    m_tot = N_DEV * m_per

    def body(x_ref, w_ref, out_ref, gath_ref, amax_ref,
             send_sems, recv_sems, am_send_sems, am_recv_sems, local_sem):
        my = lax.axis_index("i")
        left = lax.rem(my + N_DEV - 1, N_DEV)
        right = lax.rem(my + 1, N_DEV)

        cp = pltpu.make_async_copy(
            x_ref, gath_ref.at[pl.ds(my * m_per, m_per), :], local_sem)
        cp.start()

        barrier = pltpu.get_barrier_semaphore()
        for nbr in (left, right):
            _sem_signal(barrier, 1, device_id=(nbr,), device_id_type=_MESH)
        _sem_wait(barrier, 2)
        cp.wait()

        out_ref[pl.ds(my * m_per, m_per), :] = jnp.dot(
            gath_ref[pl.ds(my * m_per, m_per), :], w_ref[...],
            preferred_element_type=jnp.float32)
        amax = jnp.max(jnp.abs(out_ref[pl.ds(my * m_per, m_per), :]))

        for h in range(N_DEV - 1):
            o_s = lax.rem(my - h + N_DEV, N_DEV)
            o_r = lax.rem(my - h - 1 + N_DEV, N_DEV)
            send = pltpu.make_async_remote_copy(
                src_ref=gath_ref.at[pl.ds(o_s * m_per, m_per), :],
                dst_ref=gath_ref.at[pl.ds(o_s * m_per, m_per), :],
                send_sem=send_sems.at[h],
                recv_sem=recv_sems.at[h],
                device_id=(right,),
                device_id_type=_MESH)
            send.start()
            send.wait_send()
            recv = pltpu.make_async_remote_copy(
                src_ref=gath_ref.at[pl.ds(o_r * m_per, m_per), :],
                dst_ref=gath_ref.at[pl.ds(o_r * m_per, m_per), :],
                send_sem=send_sems.at[h],
                recv_sem=recv_sems.at[h],
                device_id=(left,),
                device_id_type=_MESH)
            recv.wait_recv()
            out_ref[pl.ds(o_r * m_per, m_per), :] = jnp.dot(
                gath_ref[pl.ds(o_r * m_per, m_per), :], w_ref[...],
                preferred_element_type=jnp.float32)
            amax = jnp.maximum(
                amax, jnp.max(jnp.abs(out_ref[pl.ds(o_r * m_per, m_per), :])))

        amax_ref[pl.ds(my, 1), :] = jnp.broadcast_to(
            amax, (1, 128)).astype(jnp.float32)
        am_sends = []
        for d in range(1, N_DEV):
            tgt = lax.rem(my + d, N_DEV)
            s = pltpu.make_async_remote_copy(
                src_ref=amax_ref.at[pl.ds(my, 1), :],
                dst_ref=amax_ref.at[pl.ds(my, 1), :],
                send_sem=am_send_sems.at[d - 1],
                recv_sem=am_recv_sems.at[d - 1],
                device_id=(tgt,),
                device_id_type=_MESH)
            s.start()
            am_sends.append(s)
        for d in range(1, N_DEV):
            src = lax.rem(my - d + N_DEV, N_DEV)
            r = pltpu.make_async_remote_copy(
                src_ref=amax_ref.at[pl.ds(src, 1), :],
                dst_ref=amax_ref.at[pl.ds(src, 1), :],
                send_sem=am_send_sems.at[d - 1],
                recv_sem=am_recv_sems.at[d - 1],
                device_id=(src,),
                device_id_type=_MESH)
            r.wait_recv()
        for s in am_sends:
            s.wait_send()

        amax_g = jnp.max(amax_ref[...])
        inv = 448.0 / amax_g
        scale = amax_g / 448.0

        def quant_block(b, _):
            y = out_ref[pl.ds(b * m_per, m_per), :]
            q = jnp.clip(y * inv, -448.0, 448.0).astype(jnp.float8_e4m3fn)
            out_ref[pl.ds(b * m_per, m_per), :] = q.astype(jnp.float32) * scale
            return _

        lax.fori_loop(0, N_DEV, quant_block, 0)

    return pl.pallas_call(
        body,
        out_shape=jax.ShapeDtypeStruct((m_tot, n_per), jnp.float32),
        in_specs=[pl.BlockSpec(memory_space=_ANY),
                  pl.BlockSpec(memory_space=pltpu.VMEM)],
        out_specs=pl.BlockSpec(memory_space=pltpu.VMEM),
        scratch_shapes=[
            pltpu.VMEM((m_tot, k), jnp.bfloat16),
            pltpu.VMEM((N_DEV, 128), jnp.float32),
            pltpu.SemaphoreType.DMA((N_DEV - 1,)),
            pltpu.SemaphoreType.DMA((N_DEV - 1,)),
            pltpu.SemaphoreType.DMA((N_DEV - 1,)),
            pltpu.SemaphoreType.DMA((N_DEV - 1,)),
            pltpu.SemaphoreType.DMA,
        ],
        compiler_params=_CompilerParams(
            collective_id=0, vmem_limit_bytes=64 * 1024 * 1024),
    )(x, w)


# device time: 217609 ns/iter; 1.9047x vs baseline; 1.9047x over previous
import jax
import jax.numpy as jnp
from jax import lax
from jax.experimental import pallas as pl
from jax.experimental.pallas import tpu as pltpu

N_DEV = 8
N_ROUNDS = 4

_sem_signal = getattr(pl, "semaphore_signal", None) or pltpu.semaphore_signal
_sem_wait = getattr(pl, "semaphore_wait", None) or pltpu.semaphore_wait
_DeviceIdType = getattr(pl, "DeviceIdType", None) or pltpu.DeviceIdType
_MESH = _DeviceIdType.MESH
_CompilerParams = getattr(pltpu, "CompilerParams", None) or pltpu.TPUCompilerParams
_ANY = getattr(pl, "ANY", None) or getattr(pltpu, "ANY", None) or (
    pltpu.MemorySpace.ANY if hasattr(pltpu, "MemorySpace")
    else pltpu.TPUMemorySpace.ANY)


def kernel(x, w_mat):
    x = x.astype(jnp.bfloat16)
    w = w_mat.astype(jnp.bfloat16)
    m_per, k = x.shape
    _, n_per = w.shape
    m_tot = N_DEV * m_per
    half = m_per // 2

    def body(x_ref, w_ref, out_ref, gath_ref, amax_ref,
             r_send_sems, r_recv_sems, l_send_sems, l_recv_sems,
             am_send_sems, am_recv_sems, local_sem):
        my = lax.axis_index("i")
        left = lax.rem(my + N_DEV - 1, N_DEV)
        right = lax.rem(my + 1, N_DEV)

        def rows(origin, which):
            if which == 0:
                return pl.ds(origin * m_per, m_per)
            if which == 1:
                return pl.ds(origin * m_per, half)
            return pl.ds(origin * m_per + half, half)

        def ring_copy(origin, which, sem_arr_s, sem_arr_r, r, dev):
            return pltpu.make_async_remote_copy(
                src_ref=gath_ref.at[rows(origin, which), :],
                dst_ref=gath_ref.at[rows(origin, which), :],
                send_sem=sem_arr_s.at[r],
                recv_sem=sem_arr_r.at[r],
                device_id=(dev,),
                device_id_type=_MESH)

        cp = pltpu.make_async_copy(
            x_ref, gath_ref.at[pl.ds(my * m_per, m_per), :], local_sem)
        cp.start()

        barrier = pltpu.get_barrier_semaphore()
        for nbr in (left, right):
            _sem_signal(barrier, 1, device_id=(nbr,), device_id_type=_MESH)
        _sem_wait(barrier, 2)
        cp.wait()

        sends = []
        s = ring_copy(my, 0, r_send_sems, r_recv_sems, 0, right)
        s.start()
        sends.append(s)
        s = ring_copy(my, 0, l_send_sems, l_recv_sems, 0, left)
        s.start()
        sends.append(s)

        out_ref[pl.ds(my * m_per, m_per), :] = jnp.dot(
            gath_ref[pl.ds(my * m_per, m_per), :], w_ref[...],
            preferred_element_type=jnp.float32)
        amax = jnp.max(jnp.abs(out_ref[pl.ds(my * m_per, m_per), :]))

        for r in range(N_ROUNDS):
            r_which = 0 if r < 3 else 1
            l_which = 0 if r < 3 else 2
            o_right = lax.rem(my - 1 - r + N_DEV, N_DEV)
            o_left = lax.rem(my + 1 + r, N_DEV)

            recv = ring_copy(o_right, r_which, r_send_sems, r_recv_sems,
                             r, left)
            recv.wait_recv()
            if r + 1 < N_ROUNDS:
                nw = 0 if r + 1 < 3 else 1
                s = ring_copy(o_right, nw, r_send_sems, r_recv_sems,
                              r + 1, right)
                s.start()
                sends.append(s)

            recv = ring_copy(o_left, l_which, l_send_sems, l_recv_sems,
                             r, right)
            recv.wait_recv()
            if r + 1 < N_ROUNDS:
                nw = 0 if r + 1 < 3 else 2
                s = ring_copy(o_left, nw, l_send_sems, l_recv_sems,
                              r + 1, left)
                s.start()
                sends.append(s)

            for origin, which in ((o_right, r_which), (o_left, l_which)):
                rsl = rows(origin, which)
                out_ref[rsl, :] = jnp.dot(
                    gath_ref[rsl, :], w_ref[...],
                    preferred_element_type=jnp.float32)
                amax = jnp.maximum(amax, jnp.max(jnp.abs(out_ref[rsl, :])))

        amax_ref[pl.ds(my, 1), :] = jnp.broadcast_to(
            amax, (1, 128)).astype(jnp.float32)
        for d in range(1, N_DEV):
            tgt = lax.rem(my + d, N_DEV)
            s = pltpu.make_async_remote_copy(
                src_ref=amax_ref.at[pl.ds(my, 1), :],
                dst_ref=amax_ref.at[pl.ds(my, 1), :],
                send_sem=am_send_sems.at[d - 1],
                recv_sem=am_recv_sems.at[d - 1],
                device_id=(tgt,),
                device_id_type=_MESH)
            s.start()
            sends.append(s)
        for d in range(1, N_DEV):
            src = lax.rem(my - d + N_DEV, N_DEV)
            rcv = pltpu.make_async_remote_copy(
                src_ref=amax_ref.at[pl.ds(src, 1), :],
                dst_ref=amax_ref.at[pl.ds(src, 1), :],
                send_sem=am_send_sems.at[d - 1],
                recv_sem=am_recv_sems.at[d - 1],
                device_id=(src,),
                device_id_type=_MESH)
            rcv.wait_recv()
        for s in sends:
            s.wait_send()

        amax_g = jnp.max(amax_ref[...])
        inv = 448.0 / amax_g
        scale = amax_g / 448.0

        def quant_block(b, _):
            y = out_ref[pl.ds(b * m_per, m_per), :]
            q = jnp.clip(y * inv, -448.0, 448.0).astype(jnp.float8_e4m3fn)
            out_ref[pl.ds(b * m_per, m_per), :] = q.astype(jnp.float32) * scale
            return _

        lax.fori_loop(0, N_DEV, quant_block, 0)

    return pl.pallas_call(
        body,
        out_shape=jax.ShapeDtypeStruct((m_tot, n_per), jnp.float32),
        in_specs=[pl.BlockSpec(memory_space=_ANY),
                  pl.BlockSpec(memory_space=pltpu.VMEM)],
        out_specs=pl.BlockSpec(memory_space=pltpu.VMEM),
        scratch_shapes=[
            pltpu.VMEM((m_tot, k), jnp.bfloat16),
            pltpu.VMEM((N_DEV, 128), jnp.float32),
            pltpu.SemaphoreType.DMA((N_ROUNDS,)),
            pltpu.SemaphoreType.DMA((N_ROUNDS,)),
            pltpu.SemaphoreType.DMA((N_ROUNDS,)),
            pltpu.SemaphoreType.DMA((N_ROUNDS,)),
            pltpu.SemaphoreType.DMA((N_DEV - 1,)),
            pltpu.SemaphoreType.DMA((N_DEV - 1,)),
            pltpu.SemaphoreType.DMA,
        ],
        compiler_params=_CompilerParams(
            collective_id=0, vmem_limit_bytes=64 * 1024 * 1024),
    )(x, w)


# device time: 210928 ns/iter; 1.9650x vs baseline; 1.0317x over previous
import jax
import jax.numpy as jnp
from jax import lax
from jax.experimental import pallas as pl
from jax.experimental.pallas import tpu as pltpu

N_DEV = 8
N_ROUNDS = 4

_sem_signal = getattr(pl, "semaphore_signal", None) or pltpu.semaphore_signal
_sem_wait = getattr(pl, "semaphore_wait", None) or pltpu.semaphore_wait
_DeviceIdType = getattr(pl, "DeviceIdType", None) or pltpu.DeviceIdType
_MESH = _DeviceIdType.MESH
_CompilerParams = getattr(pltpu, "CompilerParams", None) or pltpu.TPUCompilerParams
_ANY = getattr(pl, "ANY", None) or getattr(pltpu, "ANY", None) or (
    pltpu.MemorySpace.ANY if hasattr(pltpu, "MemorySpace")
    else pltpu.TPUMemorySpace.ANY)


def kernel(x, w_mat):
    m_per, k = x.shape
    _, n_per = w_mat.shape
    m_tot = N_DEV * m_per
    half = m_per // 2
    qw = n_per

    def body(x_ref, w_ref, out_ref, gath_ref, own_ref, chunk_ref, w_bf_ref,
             stage_ref, amax_ref,
             r_send_sems, r_recv_sems, l_send_sems, l_recv_sems,
             am_send_sems, am_recv_sems, local_sems):
        my = lax.axis_index("i")
        left = lax.rem(my + N_DEV - 1, N_DEV)
        right = lax.rem(my + 1, N_DEV)

        def rows(origin, which):
            if which == 0:
                return pl.ds(origin * m_per, m_per)
            if which == 1:
                return pl.ds(origin * m_per, half)
            return pl.ds(origin * m_per + half, half)

        barrier = pltpu.get_barrier_semaphore()
        for nbr in (left, right):
            _sem_signal(barrier, 1, device_id=(nbr,), device_id_type=_MESH)

        def x_cast(t):
            rb, cb = t % 2, t // 2
            own_ref[pl.ds(rb * half, half), pl.ds(cb * qw, qw)] = (
                stage_ref[:, pl.ds((t % 2) * qw, qw)].astype(jnp.bfloat16))

        xcps = [None, None]
        for t in range(8):
            rb, cb = t % 2, t // 2
            sl = t % 2
            if xcps[sl] is not None:
                xcps[sl].wait()
                x_cast(t - 2)
            cp = pltpu.make_async_copy(
                x_ref.at[pl.ds(rb * half, half), pl.ds(cb * qw, qw)],
                stage_ref.at[:, pl.ds(sl * qw, qw)], local_sems.at[sl])
            cp.start()
            xcps[sl] = cp
        xcps[0].wait()
        x_cast(6)
        xcps[1].wait()
        x_cast(7)

        _sem_wait(barrier, 2)

        sends = []
        for sems_s, sems_r, dev in ((r_send_sems, r_recv_sems, right),
                                    (l_send_sems, l_recv_sems, left)):
            s = pltpu.make_async_remote_copy(
                src_ref=own_ref,
                dst_ref=gath_ref.at[rows(my, 0), :],
                send_sem=sems_s.at[0],
                recv_sem=sems_r.at[0],
                device_id=(dev,),
                device_id_type=_MESH)
            s.start()
            sends.append(s)

        wb = half
        nwb = k // wb

        def w_cast(b):
            w_bf_ref[pl.ds(b * wb, wb), :] = stage_ref[
                :, pl.ds((b % 2) * n_per, n_per)].astype(jnp.bfloat16)

        wcps = [None, None]
        for b in range(nwb):
            sl = b % 2
            if wcps[sl] is not None:
                wcps[sl].wait()
                w_cast(b - 2)
            cp = pltpu.make_async_copy(
                w_ref.at[pl.ds(b * wb, wb), :],
                stage_ref.at[:, pl.ds(sl * n_per, n_per)], local_sems.at[sl])
            cp.start()
            wcps[sl] = cp
        wcps[0].wait()
        w_cast(nwb - 2)
        wcps[1].wait()
        w_cast(nwb - 1)

        out_ref[pl.ds(my * m_per, m_per), :] = jnp.dot(
            own_ref[...], w_bf_ref[...], preferred_element_type=jnp.float32)

        def ring_copy(origin, which, sem_arr_s, sem_arr_r, r, dev):
            return pltpu.make_async_remote_copy(
                src_ref=gath_ref.at[rows(origin, which), :],
                dst_ref=gath_ref.at[rows(origin, which), :],
                send_sem=sem_arr_s.at[r],
                recv_sem=sem_arr_r.at[r],
                device_id=(dev,),
                device_id_type=_MESH)

        for r in range(N_ROUNDS):
            r_which = 0 if r < 3 else 1
            l_which = 0 if r < 3 else 2
            o_right = lax.rem(my - 1 - r + N_DEV, N_DEV)
            o_left = lax.rem(my + 1 + r, N_DEV)

            recv = ring_copy(o_right, r_which, r_send_sems, r_recv_sems,
                             r, left)
            recv.wait_recv()
            if r + 1 < N_ROUNDS:
                nw = 0 if r + 1 < 3 else 1
                s = ring_copy(o_right, nw, r_send_sems, r_recv_sems,
                              r + 1, right)
                s.start()
                sends.append(s)

            recv = ring_copy(o_left, l_which, l_send_sems, l_recv_sems,
                             r, right)
            recv.wait_recv()
            if r + 1 < N_ROUNDS:
                nw = 0 if r + 1 < 3 else 2
                s = ring_copy(o_left, nw, l_send_sems, l_recv_sems,
                              r + 1, left)
                s.start()
                sends.append(s)

            nrows_r = m_per if r_which == 0 else half
            nrows_l = m_per if l_which == 0 else half
            cp_r = pltpu.make_async_copy(
                gath_ref.at[rows(o_right, r_which), :],
                chunk_ref.at[0, pl.ds(0, nrows_r), :], local_sems.at[0])
            cp_r.start()
            cp_l = pltpu.make_async_copy(
                gath_ref.at[rows(o_left, l_which), :],
                chunk_ref.at[1, pl.ds(0, nrows_l), :], local_sems.at[1])
            cp_l.start()
            cp_r.wait()
            out_ref[rows(o_right, r_which), :] = jnp.dot(
                chunk_ref[0, pl.ds(0, nrows_r), :], w_bf_ref[...],
                preferred_element_type=jnp.float32)
            cp_l.wait()
            out_ref[rows(o_left, l_which), :] = jnp.dot(
                chunk_ref[1, pl.ds(0, nrows_l), :], w_bf_ref[...],
                preferred_element_type=jnp.float32)

        def amax_block(b, acc):
            return jnp.maximum(
                acc, jnp.max(jnp.abs(out_ref[pl.ds(b * half, half), :])))

        amax = lax.fori_loop(0, 2 * N_DEV, amax_block, jnp.float32(0))

        amax_ref[pl.ds(my, 1), :] = jnp.broadcast_to(
            amax, (1, 128)).astype(jnp.float32)
        for d in range(1, N_DEV):
            tgt = lax.rem(my + d, N_DEV)
            s = pltpu.make_async_remote_copy(
                src_ref=amax_ref.at[pl.ds(my, 1), :],
                dst_ref=amax_ref.at[pl.ds(my, 1), :],
                send_sem=am_send_sems.at[d - 1],
                recv_sem=am_recv_sems.at[d - 1],
                device_id=(tgt,),
                device_id_type=_MESH)
            s.start()
            sends.append(s)
        for d in range(1, N_DEV):
            src = lax.rem(my - d + N_DEV, N_DEV)
            rcv = pltpu.make_async_remote_copy(
                src_ref=amax_ref.at[pl.ds(src, 1), :],
                dst_ref=amax_ref.at[pl.ds(src, 1), :],
                send_sem=am_send_sems.at[d - 1],
                recv_sem=am_recv_sems.at[d - 1],
                device_id=(src,),
                device_id_type=_MESH)
            rcv.wait_recv()
        for s in sends:
            s.wait_send()

        amax_g = jnp.max(amax_ref[...])
        inv = 448.0 / amax_g
        scale = amax_g / 448.0

        def quant_block(b, _):
            y = out_ref[pl.ds(b * half, half), :]
            q = jnp.clip(y * inv, -448.0, 448.0).astype(jnp.float8_e4m3fn)
            out_ref[pl.ds(b * half, half), :] = q.astype(jnp.float32) * scale
            return _

        lax.fori_loop(0, 2 * N_DEV, quant_block, 0)

    out, _ = pl.pallas_call(
        body,
        out_shape=[
            jax.ShapeDtypeStruct((m_tot, n_per), jnp.float32),
            jax.ShapeDtypeStruct((m_tot, k), jnp.bfloat16),
        ],
        in_specs=[pl.BlockSpec(memory_space=_ANY),
                  pl.BlockSpec(memory_space=_ANY)],
        out_specs=[pl.BlockSpec(memory_space=pltpu.VMEM),
                   pl.BlockSpec(memory_space=_ANY)],
        scratch_shapes=[
            pltpu.VMEM((m_per, k), jnp.bfloat16),
            pltpu.VMEM((2, m_per, k), jnp.bfloat16),
            pltpu.VMEM((k, n_per), jnp.bfloat16),
            pltpu.VMEM((half, 2 * n_per), jnp.float32),
            pltpu.VMEM((N_DEV, 128), jnp.float32),
            pltpu.SemaphoreType.DMA((N_ROUNDS,)),
            pltpu.SemaphoreType.DMA((N_ROUNDS,)),
            pltpu.SemaphoreType.DMA((N_ROUNDS,)),
            pltpu.SemaphoreType.DMA((N_ROUNDS,)),
            pltpu.SemaphoreType.DMA((N_DEV - 1,)),
            pltpu.SemaphoreType.DMA((N_DEV - 1,)),
            pltpu.SemaphoreType.DMA((2,)),
        ],
        compiler_params=_CompilerParams(
            collective_id=0, vmem_limit_bytes=64 * 1024 * 1024),
    )(x, w_mat)
    return out


# device time: 206116 ns/iter; 2.0109x vs baseline; 1.0233x over previous
import jax
import jax.numpy as jnp
from jax import lax
from jax.experimental import pallas as pl
from jax.experimental.pallas import tpu as pltpu

N_DEV = 8
N_ROUNDS = 4

_sem_signal = getattr(pl, "semaphore_signal", None) or pltpu.semaphore_signal
_sem_wait = getattr(pl, "semaphore_wait", None) or pltpu.semaphore_wait
_DeviceIdType = getattr(pl, "DeviceIdType", None) or pltpu.DeviceIdType
_MESH = _DeviceIdType.MESH
_CompilerParams = getattr(pltpu, "CompilerParams", None) or pltpu.TPUCompilerParams
_ANY = getattr(pl, "ANY", None) or getattr(pltpu, "ANY", None) or (
    pltpu.MemorySpace.ANY if hasattr(pltpu, "MemorySpace")
    else pltpu.TPUMemorySpace.ANY)


def kernel(x, w_mat):
    m_per, k = x.shape
    _, n_per = w_mat.shape
    m_tot = N_DEV * m_per
    half = m_per // 2
    qw = n_per

    def body(x_ref, w_ref, out_ref, gath_ref, own_ref, chunk_ref, w_bf_ref,
             stage_ref, amax_ref,
             r_send_sems, r_recv_sems, l_send_sems, l_recv_sems,
             am_send_sems, am_recv_sems, local_sems):
        my = lax.axis_index("i")
        left = lax.rem(my + N_DEV - 1, N_DEV)
        right = lax.rem(my + 1, N_DEV)

        def rows(origin, which):
            if which == 0:
                return pl.ds(origin * m_per, m_per)
            if which == 1:
                return pl.ds(origin * m_per, half)
            return pl.ds(origin * m_per + half, half)

        barrier = pltpu.get_barrier_semaphore()
        for nbr in (left, right):
            _sem_signal(barrier, 1, device_id=(nbr,), device_id_type=_MESH)

        cp = pltpu.make_async_copy(x_ref, stage_ref, local_sems.at[0])
        cp.start()
        cp.wait()
        own_ref[...] = stage_ref[...].astype(jnp.bfloat16)

        _sem_wait(barrier, 2)

        sends = []
        for sems_s, sems_r, dev in ((r_send_sems, r_recv_sems, right),
                                    (l_send_sems, l_recv_sems, left)):
            s = pltpu.make_async_remote_copy(
                src_ref=own_ref,
                dst_ref=gath_ref.at[rows(my, 0), :],
                send_sem=sems_s.at[0],
                recv_sem=sems_r.at[0],
                device_id=(dev,),
                device_id_type=_MESH)
            s.start()
            sends.append(s)

        wb = half
        nwb = k // wb

        def w_cast(b):
            w_bf_ref[pl.ds(b * wb, wb), :] = stage_ref[
                pl.ds(0, wb), pl.ds((b % 2) * n_per, n_per)].astype(
                    jnp.bfloat16)

        wcps = [None, None]
        for b in range(nwb):
            sl = b % 2
            if wcps[sl] is not None:
                wcps[sl].wait()
                w_cast(b - 2)
            cp = pltpu.make_async_copy(
                w_ref.at[pl.ds(b * wb, wb), :],
                stage_ref.at[pl.ds(0, wb), pl.ds(sl * n_per, n_per)],
                local_sems.at[sl])
            cp.start()
            wcps[sl] = cp
        wcps[0].wait()
        w_cast(nwb - 2)
        wcps[1].wait()
        w_cast(nwb - 1)

        out_ref[pl.ds(my * m_per, m_per), :] = jnp.dot(
            own_ref[...], w_bf_ref[...], preferred_element_type=jnp.float32)
        amax = jnp.max(jnp.abs(out_ref[pl.ds(my * m_per, m_per), :]))

        def ring_copy(origin, which, sem_arr_s, sem_arr_r, r, dev):
            return pltpu.make_async_remote_copy(
                src_ref=gath_ref.at[rows(origin, which), :],
                dst_ref=gath_ref.at[rows(origin, which), :],
                send_sem=sem_arr_s.at[r],
                recv_sem=sem_arr_r.at[r],
                device_id=(dev,),
                device_id_type=_MESH)

        for r in range(N_ROUNDS):
            r_which = 0 if r < 3 else 1
            l_which = 0 if r < 3 else 2
            o_right = lax.rem(my - 1 - r + N_DEV, N_DEV)
            o_left = lax.rem(my + 1 + r, N_DEV)

            recv = ring_copy(o_right, r_which, r_send_sems, r_recv_sems,
                             r, left)
            recv.wait_recv()
            if r + 1 < N_ROUNDS:
                nw = 0 if r + 1 < 3 else 1
                s = ring_copy(o_right, nw, r_send_sems, r_recv_sems,
                              r + 1, right)
                s.start()
                sends.append(s)

            recv = ring_copy(o_left, l_which, l_send_sems, l_recv_sems,
                             r, right)
            recv.wait_recv()
            if r + 1 < N_ROUNDS:
                nw = 0 if r + 1 < 3 else 2
                s = ring_copy(o_left, nw, l_send_sems, l_recv_sems,
                              r + 1, left)
                s.start()
                sends.append(s)

            nrows_r = m_per if r_which == 0 else half
            nrows_l = m_per if l_which == 0 else half
            cp_r = pltpu.make_async_copy(
                gath_ref.at[rows(o_right, r_which), :],
                chunk_ref.at[0, pl.ds(0, nrows_r), :], local_sems.at[0])
            cp_r.start()
            cp_l = pltpu.make_async_copy(
                gath_ref.at[rows(o_left, l_which), :],
                chunk_ref.at[1, pl.ds(0, nrows_l), :], local_sems.at[1])
            cp_l.start()
            cp_r.wait()
            out_ref[rows(o_right, r_which), :] = jnp.dot(
                chunk_ref[0, pl.ds(0, nrows_r), :], w_bf_ref[...],
                preferred_element_type=jnp.float32)
            amax = jnp.maximum(
                amax, jnp.max(jnp.abs(out_ref[rows(o_right, r_which), :])))
            cp_l.wait()
            out_ref[rows(o_left, l_which), :] = jnp.dot(
                chunk_ref[1, pl.ds(0, nrows_l), :], w_bf_ref[...],
                preferred_element_type=jnp.float32)
            amax = jnp.maximum(
                amax, jnp.max(jnp.abs(out_ref[rows(o_left, l_which), :])))

        amax_ref[pl.ds(my, 1), :] = jnp.broadcast_to(
            amax, (1, 128)).astype(jnp.float32)
        for d in range(1, N_DEV):
            tgt = lax.rem(my + d, N_DEV)
            s = pltpu.make_async_remote_copy(
                src_ref=amax_ref.at[pl.ds(my, 1), :],
                dst_ref=amax_ref.at[pl.ds(my, 1), :],
                send_sem=am_send_sems.at[d - 1],
                recv_sem=am_recv_sems.at[d - 1],
                device_id=(tgt,),
                device_id_type=_MESH)
            s.start()
            sends.append(s)
        for d in range(1, N_DEV):
            src = lax.rem(my - d + N_DEV, N_DEV)
            rcv = pltpu.make_async_remote_copy(
                src_ref=amax_ref.at[pl.ds(src, 1), :],
                dst_ref=amax_ref.at[pl.ds(src, 1), :],
                send_sem=am_send_sems.at[d - 1],
                recv_sem=am_recv_sems.at[d - 1],
                device_id=(src,),
                device_id_type=_MESH)
            rcv.wait_recv()
        for s in sends:
            s.wait_send()

        amax_g = jnp.max(amax_ref[...])
        inv = 448.0 / amax_g
        scale = amax_g / 448.0

        def quant_block(b, _):
            y = out_ref[pl.ds(b * half, half), :]
            q = jnp.clip(y * inv, -448.0, 448.0).astype(jnp.float8_e4m3fn)
            out_ref[pl.ds(b * half, half), :] = q.astype(jnp.float32) * scale
            return _

        lax.fori_loop(0, 2 * N_DEV, quant_block, 0)

    out, _ = pl.pallas_call(
        body,
        out_shape=[
            jax.ShapeDtypeStruct((m_tot, n_per), jnp.float32),
            jax.ShapeDtypeStruct((m_tot, k), jnp.bfloat16),
        ],
        in_specs=[pl.BlockSpec(memory_space=_ANY),
                  pl.BlockSpec(memory_space=_ANY)],
        out_specs=[pl.BlockSpec(memory_space=pltpu.VMEM),
                   pl.BlockSpec(memory_space=_ANY)],
        scratch_shapes=[
            pltpu.VMEM((m_per, k), jnp.bfloat16),
            pltpu.VMEM((2, m_per, k), jnp.bfloat16),
            pltpu.VMEM((k, n_per), jnp.bfloat16),
            pltpu.VMEM((m_per, k), jnp.float32),
            pltpu.VMEM((N_DEV, 128), jnp.float32),
            pltpu.SemaphoreType.DMA((N_ROUNDS,)),
            pltpu.SemaphoreType.DMA((N_ROUNDS,)),
            pltpu.SemaphoreType.DMA((N_ROUNDS,)),
            pltpu.SemaphoreType.DMA((N_ROUNDS,)),
            pltpu.SemaphoreType.DMA((N_DEV - 1,)),
            pltpu.SemaphoreType.DMA((N_DEV - 1,)),
            pltpu.SemaphoreType.DMA((2,)),
        ],
        compiler_params=_CompilerParams(
            collective_id=0, vmem_limit_bytes=64 * 1024 * 1024),
    )(x, w_mat)
    return out
